# baseline (speedup 1.0000x reference)
import functools

import jax
import jax.numpy as jnp
import numpy as np
from jax.experimental import pallas as pl
from jax.experimental.pallas import tpu as pltpu

N_POINTS = 16384
N1, N2, N3 = 4096, 1024, 256
K_SA = 16
K_FP = 3


def _mlp_res(x, p):
    h = jnp.maximum(x @ p['w1'].T + p['b1'], 0.0)
    h = h @ p['w2'].T + p['b2']
    sc = (x @ p['ws'].T + p['bs']) if 'ws' in p else x
    return h + sc


def _fps_x(pos, n_out):
    d0 = jnp.sum((pos - pos[0]) ** 2, axis=-1)
    idx0 = jnp.zeros((n_out,), dtype=jnp.int32)

    def body(i, state):
        dists, idx = state
        nxt = jnp.argmax(dists).astype(jnp.int32)
        idx = idx.at[i].set(nxt)
        d = jnp.sum((pos - pos[nxt]) ** 2, axis=-1)
        return (jnp.minimum(dists, d), idx)

    _, idx = jax.lax.fori_loop(1, n_out, body, (d0, idx0))
    return idx


def _knn_x(pos_x, pos_y, k):
    d2 = (jnp.sum(pos_y ** 2, axis=1)[:, None] + jnp.sum(pos_x ** 2, axis=1)[None, :]
          - 2.0 * (pos_y @ pos_x.T))
    d2 = jnp.maximum(d2, 0.0)
    _, nbr = jax.lax.top_k(-d2, k)
    return nbr


def _fp_w_x(pos_s, pos_t):
    nbr = _knn_x(pos_s, pos_t, K_FP)
    diff = pos_s[nbr] - pos_t[:, None, :]
    d2 = jnp.sum(diff ** 2, axis=-1)
    w = 1.0 / jnp.maximum(d2, 1e-16)
    return nbr, w


def _sa_conv_x(x, pos_s, pos_t, nbr, p):
    nt, k = nbr.shape
    loops = jnp.arange(nt, dtype=nbr.dtype)
    src = jnp.concatenate([nbr.reshape(-1), loops])
    dst = jnp.concatenate([jnp.repeat(loops, k), loops])
    rel = pos_s[src] - pos_t[dst]
    h = rel if x is None else jnp.concatenate([x[src], rel], axis=1)
    m = jnp.maximum(_mlp_res(h, p), 0.0)
    return jax.ops.segment_max(m, dst, num_segments=nt)


def _fp_apply_x(x_t, x_s, nbr, w, p):
    interp = jnp.sum(x_s[nbr] * w[..., None], axis=1) / jnp.sum(w, axis=1, keepdims=True)
    comb = interp if x_t is None else jnp.concatenate([x_t, interp], axis=1)
    return jnp.maximum(_mlp_res(comb, p), 0.0)


def _final_body(fin_ref, w1_ref, b1_ref, w2_ref, b2_ref, out_ref):
    h = jnp.maximum(fin_ref[...] @ w1_ref[...].T + b1_ref[...], 0.0)
    out_ref[...] = h @ w2_ref[...].T + b2_ref[...]


def _final_pallas(fin, w1, b1, w2, b2):
    n, d = fin.shape
    out_d = w2.shape[0]
    blk = 2048
    return pl.pallas_call(
        _final_body,
        grid=(n // blk,),
        in_specs=[
            pl.BlockSpec((blk, d), lambda i: (i, 0)),
            pl.BlockSpec((d, d), lambda i: (0, 0)),
            pl.BlockSpec((d,), lambda i: (0,)),
            pl.BlockSpec((out_d, d), lambda i: (0, 0)),
            pl.BlockSpec((out_d,), lambda i: (0,)),
        ],
        out_specs=pl.BlockSpec((blk, out_d), lambda i: (i, 0)),
        out_shape=jax.ShapeDtypeStruct((n, out_d), jnp.float32),
    )(fin, w1, b1, w2, b2)


def kernel(z, c, query_pos, query_pos_batch, params):
    pos0 = query_pos
    idx1 = _fps_x(pos0, N1)
    pos1 = pos0[idx1]
    nbr_sa1 = _knn_x(pos0, pos1, K_SA)
    idx2 = _fps_x(pos1, N2)
    pos2 = pos1[idx2]
    nbr_sa2 = _knn_x(pos1, pos2, K_SA)
    idx3 = _fps_x(pos2, N3)
    pos3 = pos2[idx3]
    nbr_sa3 = _knn_x(pos2, pos3, K_SA)
    batch3 = query_pos_batch[idx1][idx2][idx3]
    nbr_fp1, w_fp1 = _fp_w_x(pos3, pos3)
    nbr_fp2, w_fp2 = _fp_w_x(pos3, pos2)
    nbr_fp3, w_fp3 = _fp_w_x(pos2, pos1)
    nbr_fp4, w_fp4 = _fp_w_x(pos1, pos0)

    x1 = _sa_conv_x(None, pos0, pos1, nbr_sa1, params['sa1'])
    x2 = _sa_conv_x(x1, pos1, pos2, nbr_sa2, params['sa2'])
    x3 = _sa_conv_x(x2, pos2, pos3, nbr_sa3, params['sa3'])
    bott_in = jnp.concatenate([x3, z[batch3], c[batch3]], axis=1)
    bott = jnp.maximum(_mlp_res(bott_in, params['bottleneck']), 0.0)
    up3 = _fp_apply_x(x3, bott, nbr_fp1, w_fp1, params['fp1'])
    up2 = _fp_apply_x(x2, up3, nbr_fp2, w_fp2, params['fp2'])
    up1 = _fp_apply_x(x1, up2, nbr_fp3, w_fp3, params['fp3'])
    fin = _fp_apply_x(None, up1, nbr_fp4, w_fp4, params['fp4'])
    return _final_pallas(fin, params['final_w1'], params['final_b1'],
                         params['final_w2'], params['final_b2'])


# speedup vs baseline: 4.1469x; 4.1469x over previous
import functools

import jax
import jax.numpy as jnp
import numpy as np
from jax.experimental import pallas as pl
from jax.experimental.pallas import tpu as pltpu

N_POINTS = 16384
N1, N2, N3 = 4096, 1024, 256
K_SA = 16
K_FP = 3


def _mlp_res(x, p):
    h = jnp.maximum(x @ p['w1'].T + p['b1'], 0.0)
    h = h @ p['w2'].T + p['b2']
    sc = (x @ p['ws'].T + p['bs']) if 'ws' in p else x
    return h + sc


def _fps_level(X, Y, Z, n, m):
    cols = n // 8
    ocols = m // 8
    lin = (jax.lax.broadcasted_iota(jnp.int32, (8, cols), 0) * cols
           + jax.lax.broadcasted_iota(jnp.int32, (8, cols), 1))
    olin = (jax.lax.broadcasted_iota(jnp.int32, (8, ocols), 0) * ocols
            + jax.lax.broadcasted_iota(jnp.int32, (8, ocols), 1))
    x0 = X[0, 0]
    y0 = Y[0, 0]
    z0 = Z[0, 0]
    dx = X - x0
    dy = Y - y0
    dz = Z - z0
    dists = (dx * dx + dy * dy) + dz * dz
    zf = jnp.zeros((8, ocols), jnp.float32)
    idxp = jnp.zeros((8, ocols), jnp.int32)
    sel0 = olin == 0
    X1 = jnp.where(sel0, x0, zf)
    Y1 = jnp.where(sel0, y0, zf)
    Z1 = jnp.where(sel0, z0, zf)

    def body(i, st):
        dists, idxp, X1, Y1, Z1 = st
        mval = jnp.max(dists)
        nxt = jnp.min(jnp.where(dists == mval, lin, jnp.int32(2 ** 30)))
        sel = lin == nxt
        sx = jnp.sum(jnp.where(sel, X, 0.0))
        sy = jnp.sum(jnp.where(sel, Y, 0.0))
        sz = jnp.sum(jnp.where(sel, Z, 0.0))
        ddx = X - sx
        ddy = Y - sy
        ddz = Z - sz
        d = (ddx * ddx + ddy * ddy) + ddz * ddz
        dists = jnp.minimum(dists, d)
        om = olin == i
        idxp = jnp.where(om, nxt, idxp)
        X1 = jnp.where(om, sx, X1)
        Y1 = jnp.where(om, sy, Y1)
        Z1 = jnp.where(om, sz, Z1)
        return dists, idxp, X1, Y1, Z1

    _, idxp, X1, Y1, Z1 = jax.lax.fori_loop(
        1, m, body, (dists, idxp, X1, Y1, Z1))
    return idxp, X1, Y1, Z1


def _fps_body(px_ref, py_ref, pz_ref,
              i1_ref, x1_ref, y1_ref, z1_ref,
              i2_ref, x2_ref, y2_ref, z2_ref,
              i3_ref, x3_ref, y3_ref, z3_ref):
    X0, Y0, Z0 = px_ref[...], py_ref[...], pz_ref[...]
    i1, X1, Y1, Z1 = _fps_level(X0, Y0, Z0, N_POINTS, N1)
    i2, X2, Y2, Z2 = _fps_level(X1, Y1, Z1, N1, N2)
    i3, X3, Y3, Z3 = _fps_level(X2, Y2, Z2, N2, N3)
    i1_ref[...], x1_ref[...], y1_ref[...], z1_ref[...] = i1, X1, Y1, Z1
    i2_ref[...], x2_ref[...], y2_ref[...], z2_ref[...] = i2, X2, Y2, Z2
    i3_ref[...], x3_ref[...], y3_ref[...], z3_ref[...] = i3, X3, Y3, Z3


def _fps_pallas(pos0):
    planes = [pos0[:, c].reshape(8, -1) for c in range(3)]
    f32, i32 = jnp.float32, jnp.int32
    outs = pl.pallas_call(
        _fps_body,
        out_shape=[
            jax.ShapeDtypeStruct((8, N1 // 8), i32),
            jax.ShapeDtypeStruct((8, N1 // 8), f32),
            jax.ShapeDtypeStruct((8, N1 // 8), f32),
            jax.ShapeDtypeStruct((8, N1 // 8), f32),
            jax.ShapeDtypeStruct((8, N2 // 8), i32),
            jax.ShapeDtypeStruct((8, N2 // 8), f32),
            jax.ShapeDtypeStruct((8, N2 // 8), f32),
            jax.ShapeDtypeStruct((8, N2 // 8), f32),
            jax.ShapeDtypeStruct((8, N3 // 8), i32),
            jax.ShapeDtypeStruct((8, N3 // 8), f32),
            jax.ShapeDtypeStruct((8, N3 // 8), f32),
            jax.ShapeDtypeStruct((8, N3 // 8), f32),
        ],
    )(*planes)
    i1, x1, y1, z1, i2, x2, y2, z2, i3, x3, y3, z3 = outs

    def pk(x, y, z):
        return jnp.stack([x.reshape(-1), y.reshape(-1), z.reshape(-1)], axis=1)

    return (i1.reshape(-1), pk(x1, y1, z1),
            i2.reshape(-1), pk(x2, y2, z2),
            i3.reshape(-1), pk(x3, y3, z3))


def _knn_x(pos_x, pos_y, k):
    d2 = (jnp.sum(pos_y ** 2, axis=1)[:, None] + jnp.sum(pos_x ** 2, axis=1)[None, :]
          - 2.0 * (pos_y @ pos_x.T))
    d2 = jnp.maximum(d2, 0.0)
    _, nbr = jax.lax.top_k(-d2, k)
    return nbr


def _fp_w_x(pos_s, pos_t):
    nbr = _knn_x(pos_s, pos_t, K_FP)
    diff = pos_s[nbr] - pos_t[:, None, :]
    d2 = jnp.sum(diff ** 2, axis=-1)
    w = 1.0 / jnp.maximum(d2, 1e-16)
    return nbr, w


def _sa_conv_x(x, pos_s, pos_t, nbr, p):
    nt, k = nbr.shape
    loops = jnp.arange(nt, dtype=nbr.dtype)
    src = jnp.concatenate([nbr.reshape(-1), loops])
    dst = jnp.concatenate([jnp.repeat(loops, k), loops])
    rel = pos_s[src] - pos_t[dst]
    h = rel if x is None else jnp.concatenate([x[src], rel], axis=1)
    m = jnp.maximum(_mlp_res(h, p), 0.0)
    return jax.ops.segment_max(m, dst, num_segments=nt)


def _fp_apply_x(x_t, x_s, nbr, w, p):
    interp = jnp.sum(x_s[nbr] * w[..., None], axis=1) / jnp.sum(w, axis=1, keepdims=True)
    comb = interp if x_t is None else jnp.concatenate([x_t, interp], axis=1)
    return jnp.maximum(_mlp_res(comb, p), 0.0)


def _final_body(fin_ref, w1_ref, b1_ref, w2_ref, b2_ref, out_ref):
    h = jnp.maximum(fin_ref[...] @ w1_ref[...].T + b1_ref[...], 0.0)
    out_ref[...] = h @ w2_ref[...].T + b2_ref[...]


def _final_pallas(fin, w1, b1, w2, b2):
    n, d = fin.shape
    out_d = w2.shape[0]
    blk = 2048
    return pl.pallas_call(
        _final_body,
        grid=(n // blk,),
        in_specs=[
            pl.BlockSpec((blk, d), lambda i: (i, 0)),
            pl.BlockSpec((d, d), lambda i: (0, 0)),
            pl.BlockSpec((d,), lambda i: (0,)),
            pl.BlockSpec((out_d, d), lambda i: (0, 0)),
            pl.BlockSpec((out_d,), lambda i: (0,)),
        ],
        out_specs=pl.BlockSpec((blk, out_d), lambda i: (i, 0)),
        out_shape=jax.ShapeDtypeStruct((n, out_d), jnp.float32),
    )(fin, w1, b1, w2, b2)


def kernel(z, c, query_pos, query_pos_batch, params):
    pos0 = query_pos
    idx1, pos1, idx2, pos2, idx3, pos3 = _fps_pallas(pos0)
    nbr_sa1 = _knn_x(pos0, pos1, K_SA)
    nbr_sa2 = _knn_x(pos1, pos2, K_SA)
    nbr_sa3 = _knn_x(pos2, pos3, K_SA)
    batch3 = query_pos_batch[idx1][idx2][idx3]
    nbr_fp1, w_fp1 = _fp_w_x(pos3, pos3)
    nbr_fp2, w_fp2 = _fp_w_x(pos3, pos2)
    nbr_fp3, w_fp3 = _fp_w_x(pos2, pos1)
    nbr_fp4, w_fp4 = _fp_w_x(pos1, pos0)

    x1 = _sa_conv_x(None, pos0, pos1, nbr_sa1, params['sa1'])
    x2 = _sa_conv_x(x1, pos1, pos2, nbr_sa2, params['sa2'])
    x3 = _sa_conv_x(x2, pos2, pos3, nbr_sa3, params['sa3'])
    bott_in = jnp.concatenate([x3, z[batch3], c[batch3]], axis=1)
    bott = jnp.maximum(_mlp_res(bott_in, params['bottleneck']), 0.0)
    up3 = _fp_apply_x(x3, bott, nbr_fp1, w_fp1, params['fp1'])
    up2 = _fp_apply_x(x2, up3, nbr_fp2, w_fp2, params['fp2'])
    up1 = _fp_apply_x(x1, up2, nbr_fp3, w_fp3, params['fp3'])
    fin = _fp_apply_x(None, up1, nbr_fp4, w_fp4, params['fp4'])
    return _final_pallas(fin, params['final_w1'], params['final_b1'],
                         params['final_w2'], params['final_b2'])


# speedup vs baseline: 8.3296x; 2.0087x over previous
import functools

import jax
import jax.numpy as jnp
import numpy as np
from jax.experimental import pallas as pl
from jax.experimental.pallas import tpu as pltpu

N_POINTS = 16384
N1, N2, N3 = 4096, 1024, 256
K_SA = 16
K_FP = 3


def _mlp_res(x, p):
    h = jnp.maximum(x @ p['w1'].T + p['b1'], 0.0)
    h = h @ p['w2'].T + p['b2']
    sc = (x @ p['ws'].T + p['bs']) if 'ws' in p else x
    return h + sc


def _fps_level(X, Y, Z, n, m):
    cols = n // 8
    ocols = m // 8
    lin = (jax.lax.broadcasted_iota(jnp.int32, (8, cols), 0) * cols
           + jax.lax.broadcasted_iota(jnp.int32, (8, cols), 1))
    olin = (jax.lax.broadcasted_iota(jnp.int32, (8, ocols), 0) * ocols
            + jax.lax.broadcasted_iota(jnp.int32, (8, ocols), 1))
    x0 = X[0, 0]
    y0 = Y[0, 0]
    z0 = Z[0, 0]
    dx = X - x0
    dy = Y - y0
    dz = Z - z0
    dists = (dx * dx + dy * dy) + dz * dz
    zf = jnp.zeros((8, ocols), jnp.float32)
    idxp = jnp.zeros((8, ocols), jnp.int32)
    sel0 = olin == 0
    X1 = jnp.where(sel0, x0, zf)
    Y1 = jnp.where(sel0, y0, zf)
    Z1 = jnp.where(sel0, z0, zf)

    def body(i, st):
        dists, idxp, X1, Y1, Z1 = st
        mval = jnp.max(dists)
        nxt = jnp.min(jnp.where(dists == mval, lin, jnp.int32(2 ** 30)))
        sel = lin == nxt
        sx = jnp.sum(jnp.where(sel, X, 0.0))
        sy = jnp.sum(jnp.where(sel, Y, 0.0))
        sz = jnp.sum(jnp.where(sel, Z, 0.0))
        ddx = X - sx
        ddy = Y - sy
        ddz = Z - sz
        d = (ddx * ddx + ddy * ddy) + ddz * ddz
        dists = jnp.minimum(dists, d)
        om = olin == i
        idxp = jnp.where(om, nxt, idxp)
        X1 = jnp.where(om, sx, X1)
        Y1 = jnp.where(om, sy, Y1)
        Z1 = jnp.where(om, sz, Z1)
        return dists, idxp, X1, Y1, Z1

    _, idxp, X1, Y1, Z1 = jax.lax.fori_loop(
        1, m, body, (dists, idxp, X1, Y1, Z1))
    return idxp, X1, Y1, Z1


def _fps_body(px_ref, py_ref, pz_ref,
              i1_ref, x1_ref, y1_ref, z1_ref,
              i2_ref, x2_ref, y2_ref, z2_ref,
              i3_ref, x3_ref, y3_ref, z3_ref):
    X0, Y0, Z0 = px_ref[...], py_ref[...], pz_ref[...]
    i1, X1, Y1, Z1 = _fps_level(X0, Y0, Z0, N_POINTS, N1)
    i2, X2, Y2, Z2 = _fps_level(X1, Y1, Z1, N1, N2)
    i3, X3, Y3, Z3 = _fps_level(X2, Y2, Z2, N2, N3)
    i1_ref[...], x1_ref[...], y1_ref[...], z1_ref[...] = i1, X1, Y1, Z1
    i2_ref[...], x2_ref[...], y2_ref[...], z2_ref[...] = i2, X2, Y2, Z2
    i3_ref[...], x3_ref[...], y3_ref[...], z3_ref[...] = i3, X3, Y3, Z3


def _fps_pallas(pos0):
    planes = [pos0[:, c].reshape(8, -1) for c in range(3)]
    f32, i32 = jnp.float32, jnp.int32
    outs = pl.pallas_call(
        _fps_body,
        out_shape=[
            jax.ShapeDtypeStruct((8, N1 // 8), i32),
            jax.ShapeDtypeStruct((8, N1 // 8), f32),
            jax.ShapeDtypeStruct((8, N1 // 8), f32),
            jax.ShapeDtypeStruct((8, N1 // 8), f32),
            jax.ShapeDtypeStruct((8, N2 // 8), i32),
            jax.ShapeDtypeStruct((8, N2 // 8), f32),
            jax.ShapeDtypeStruct((8, N2 // 8), f32),
            jax.ShapeDtypeStruct((8, N2 // 8), f32),
            jax.ShapeDtypeStruct((8, N3 // 8), i32),
            jax.ShapeDtypeStruct((8, N3 // 8), f32),
            jax.ShapeDtypeStruct((8, N3 // 8), f32),
            jax.ShapeDtypeStruct((8, N3 // 8), f32),
        ],
    )(*planes)
    i1, x1, y1, z1, i2, x2, y2, z2, i3, x3, y3, z3 = outs

    def pk(x, y, z):
        return jnp.stack([x.reshape(-1), y.reshape(-1), z.reshape(-1)], axis=1)

    def pln(x, y, z):
        return (x.reshape(1, -1), y.reshape(1, -1), z.reshape(1, -1))

    return (i1.reshape(-1), pk(x1, y1, z1), pln(x1, y1, z1),
            i2.reshape(-1), pk(x2, y2, z2), pln(x2, y2, z2),
            i3.reshape(-1), pk(x3, y3, z3), pln(x3, y3, z3))


def _knn_body(pt_ref, s_ref, nbr_ref, d2_ref, *, k, ns, tb, need_d2):
    pt = pt_ref[...]
    s = s_ref[...]
    xt = pt[:, 0:1]
    yt = pt[:, 1:2]
    zt = pt[:, 2:3]
    sx = s[0:1, :]
    sy = s[1:2, :]
    sz = s[2:3, :]
    nx = (sx * sx + sy * sy) + sz * sz
    ny = (xt * xt + yt * yt) + zt * zt
    mm = jnp.dot(pt, s, preferred_element_type=jnp.float32)
    d2q = (ny + nx) - 2.0 * mm
    d2q = jnp.maximum(d2q, 0.0)
    if need_d2:
        dx = xt - sx
        dy = yt - sy
        dz = zt - sz
        d2d = (dx * dx + dy * dy) + dz * dz
    lin = jax.lax.broadcasted_iota(jnp.int32, (tb, ns), 1)
    lane = jax.lax.broadcasted_iota(jnp.int32, (tb, 128), 1)
    nbr = jnp.zeros((tb, 128), jnp.int32)
    dsel = jnp.zeros((tb, 128), jnp.float32)
    for j in range(k):
        m = jnp.min(d2q, axis=1, keepdims=True)
        idx = jnp.min(jnp.where(d2q == m, lin, jnp.int32(2 ** 30)),
                      axis=1, keepdims=True)
        nbr = jnp.where(lane == j, idx, nbr)
        if need_d2:
            dj = jnp.min(jnp.where(lin == idx, d2d, jnp.inf),
                         axis=1, keepdims=True)
        else:
            dj = m
        dsel = jnp.where(lane == j, dj, dsel)
        d2q = jnp.where(lin == idx, jnp.inf, d2q)
    nbr_ref[...] = nbr
    d2_ref[...] = dsel


def _knn_pallas(pos_t, s_mat, k, need_d2=False):
    nt = pos_t.shape[0]
    ns = s_mat.shape[1]
    tb = min(nt, 128)
    body = functools.partial(_knn_body, k=k, ns=ns, tb=tb, need_d2=need_d2)
    nbr, d2 = pl.pallas_call(
        body,
        grid=(nt // tb,),
        in_specs=[
            pl.BlockSpec((tb, 3), lambda i: (i, 0)),
            pl.BlockSpec((3, ns), lambda i: (0, 0)),
        ],
        out_specs=[
            pl.BlockSpec((tb, 128), lambda i: (i, 0)),
            pl.BlockSpec((tb, 128), lambda i: (i, 0)),
        ],
        out_shape=[
            jax.ShapeDtypeStruct((nt, 128), jnp.int32),
            jax.ShapeDtypeStruct((nt, 128), jnp.float32),
        ],
    )(pos_t, s_mat)
    return nbr, d2


def _sa_conv_x(x, pos_s, pos_t, nbr, p):
    nt, k = nbr.shape
    loops = jnp.arange(nt, dtype=nbr.dtype)
    src = jnp.concatenate([nbr.reshape(-1), loops])
    dst = jnp.concatenate([jnp.repeat(loops, k), loops])
    rel = pos_s[src] - pos_t[dst]
    h = rel if x is None else jnp.concatenate([x[src], rel], axis=1)
    m = jnp.maximum(_mlp_res(h, p), 0.0)
    return jax.ops.segment_max(m, dst, num_segments=nt)


def _fp_apply_x(x_t, x_s, nbr, w, p):
    interp = jnp.sum(x_s[nbr] * w[..., None], axis=1) / jnp.sum(w, axis=1, keepdims=True)
    comb = interp if x_t is None else jnp.concatenate([x_t, interp], axis=1)
    return jnp.maximum(_mlp_res(comb, p), 0.0)


def _final_body(fin_ref, w1_ref, b1_ref, w2_ref, b2_ref, out_ref):
    h = jnp.maximum(fin_ref[...] @ w1_ref[...].T + b1_ref[...], 0.0)
    out_ref[...] = h @ w2_ref[...].T + b2_ref[...]


def _final_pallas(fin, w1, b1, w2, b2):
    n, d = fin.shape
    out_d = w2.shape[0]
    blk = 2048
    return pl.pallas_call(
        _final_body,
        grid=(n // blk,),
        in_specs=[
            pl.BlockSpec((blk, d), lambda i: (i, 0)),
            pl.BlockSpec((d, d), lambda i: (0, 0)),
            pl.BlockSpec((d,), lambda i: (0,)),
            pl.BlockSpec((out_d, d), lambda i: (0, 0)),
            pl.BlockSpec((out_d,), lambda i: (0,)),
        ],
        out_specs=pl.BlockSpec((blk, out_d), lambda i: (i, 0)),
        out_shape=jax.ShapeDtypeStruct((n, out_d), jnp.float32),
    )(fin, w1, b1, w2, b2)


def kernel(z, c, query_pos, query_pos_batch, params):
    pos0 = query_pos
    (idx1, pos1, pln1, idx2, pos2, pln2,
     idx3, pos3, pln3) = _fps_pallas(pos0)
    s0 = pos0.T
    s1 = jnp.concatenate(pln1, axis=0)
    s2 = jnp.concatenate(pln2, axis=0)
    s3 = jnp.concatenate(pln3, axis=0)
    nbr_sa1 = _knn_pallas(pos1, s0, K_SA)[0][:, :K_SA]
    nbr_sa2 = _knn_pallas(pos2, s1, K_SA)[0][:, :K_SA]
    nbr_sa3 = _knn_pallas(pos3, s2, K_SA)[0][:, :K_SA]
    batch3 = query_pos_batch[idx1][idx2][idx3]

    def fpw(pos_t, s_mat):
        nbr, d2 = _knn_pallas(pos_t, s_mat, K_FP, need_d2=True)
        return nbr[:, :K_FP], 1.0 / jnp.maximum(d2[:, :K_FP], 1e-16)

    nbr_fp1, w_fp1 = fpw(pos3, s3)
    nbr_fp2, w_fp2 = fpw(pos2, s3)
    nbr_fp3, w_fp3 = fpw(pos1, s2)
    nbr_fp4, w_fp4 = fpw(pos0, s1)

    x1 = _sa_conv_x(None, pos0, pos1, nbr_sa1, params['sa1'])
    x2 = _sa_conv_x(x1, pos1, pos2, nbr_sa2, params['sa2'])
    x3 = _sa_conv_x(x2, pos2, pos3, nbr_sa3, params['sa3'])
    bott_in = jnp.concatenate([x3, z[batch3], c[batch3]], axis=1)
    bott = jnp.maximum(_mlp_res(bott_in, params['bottleneck']), 0.0)
    up3 = _fp_apply_x(x3, bott, nbr_fp1, w_fp1, params['fp1'])
    up2 = _fp_apply_x(x2, up3, nbr_fp2, w_fp2, params['fp2'])
    up1 = _fp_apply_x(x1, up2, nbr_fp3, w_fp3, params['fp3'])
    fin = _fp_apply_x(None, up1, nbr_fp4, w_fp4, params['fp4'])
    return _final_pallas(fin, params['final_w1'], params['final_b1'],
                         params['final_w2'], params['final_b2'])


# speedup vs baseline: 10.2285x; 1.2280x over previous
import functools

import jax
import jax.numpy as jnp
from jax.experimental import pallas as pl
from jax.experimental.pallas import tpu as pltpu
from jax.experimental.pallas import tpu_sc as plsc

N_POINTS = 16384
N1, N2, N3 = 4096, 1024, 256
K_SA = 16
K_FP = 3


def _fps_level(X, Y, Z, n, m):
    cols = n // 8
    ocols = m // 8
    lin = (jax.lax.broadcasted_iota(jnp.int32, (8, cols), 0) * cols
           + jax.lax.broadcasted_iota(jnp.int32, (8, cols), 1))
    olin = (jax.lax.broadcasted_iota(jnp.int32, (8, ocols), 0) * ocols
            + jax.lax.broadcasted_iota(jnp.int32, (8, ocols), 1))
    x0 = X[0, 0]
    y0 = Y[0, 0]
    z0 = Z[0, 0]
    dx = X - x0
    dy = Y - y0
    dz = Z - z0
    dists = (dx * dx + dy * dy) + dz * dz
    zf = jnp.zeros((8, ocols), jnp.float32)
    idxp = jnp.zeros((8, ocols), jnp.int32)
    sel0 = olin == 0
    X1 = jnp.where(sel0, x0, zf)
    Y1 = jnp.where(sel0, y0, zf)
    Z1 = jnp.where(sel0, z0, zf)

    def body(i, st):
        dists, idxp, X1, Y1, Z1 = st
        mval = jnp.max(dists)
        nxt = jnp.min(jnp.where(dists == mval, lin, jnp.int32(2 ** 30)))
        sel = lin == nxt
        sx = jnp.sum(jnp.where(sel, X, 0.0))
        sy = jnp.sum(jnp.where(sel, Y, 0.0))
        sz = jnp.sum(jnp.where(sel, Z, 0.0))
        ddx = X - sx
        ddy = Y - sy
        ddz = Z - sz
        d = (ddx * ddx + ddy * ddy) + ddz * ddz
        dists = jnp.minimum(dists, d)
        om = olin == i
        idxp = jnp.where(om, nxt, idxp)
        X1 = jnp.where(om, sx, X1)
        Y1 = jnp.where(om, sy, Y1)
        Z1 = jnp.where(om, sz, Z1)
        return dists, idxp, X1, Y1, Z1

    _, idxp, X1, Y1, Z1 = jax.lax.fori_loop(
        1, m, body, (dists, idxp, X1, Y1, Z1))
    return idxp, X1, Y1, Z1


def _fps_body(px_ref, py_ref, pz_ref,
              i1_ref, x1_ref, y1_ref, z1_ref,
              i2_ref, x2_ref, y2_ref, z2_ref,
              i3_ref, x3_ref, y3_ref, z3_ref):
    X0, Y0, Z0 = px_ref[...], py_ref[...], pz_ref[...]
    i1, X1, Y1, Z1 = _fps_level(X0, Y0, Z0, N_POINTS, N1)
    i2, X2, Y2, Z2 = _fps_level(X1, Y1, Z1, N1, N2)
    i3, X3, Y3, Z3 = _fps_level(X2, Y2, Z2, N2, N3)
    i1_ref[...], x1_ref[...], y1_ref[...], z1_ref[...] = i1, X1, Y1, Z1
    i2_ref[...], x2_ref[...], y2_ref[...], z2_ref[...] = i2, X2, Y2, Z2
    i3_ref[...], x3_ref[...], y3_ref[...], z3_ref[...] = i3, X3, Y3, Z3


def _fps_pallas(pos0):
    planes = [pos0[:, c].reshape(8, -1) for c in range(3)]
    f32, i32 = jnp.float32, jnp.int32
    outs = pl.pallas_call(
        _fps_body,
        out_shape=[
            jax.ShapeDtypeStruct((8, N1 // 8), i32),
            jax.ShapeDtypeStruct((8, N1 // 8), f32),
            jax.ShapeDtypeStruct((8, N1 // 8), f32),
            jax.ShapeDtypeStruct((8, N1 // 8), f32),
            jax.ShapeDtypeStruct((8, N2 // 8), i32),
            jax.ShapeDtypeStruct((8, N2 // 8), f32),
            jax.ShapeDtypeStruct((8, N2 // 8), f32),
            jax.ShapeDtypeStruct((8, N2 // 8), f32),
            jax.ShapeDtypeStruct((8, N3 // 8), i32),
            jax.ShapeDtypeStruct((8, N3 // 8), f32),
            jax.ShapeDtypeStruct((8, N3 // 8), f32),
            jax.ShapeDtypeStruct((8, N3 // 8), f32),
        ],
    )(*planes)
    i1, x1, y1, z1, i2, x2, y2, z2, i3, x3, y3, z3 = outs

    def pk(x, y, z):
        return jnp.stack([x.reshape(-1), y.reshape(-1), z.reshape(-1)], axis=1)

    def pln(x, y, z):
        return (x.reshape(1, -1), y.reshape(1, -1), z.reshape(1, -1))

    return (i1.reshape(-1), pk(x1, y1, z1), pln(x1, y1, z1),
            i2.reshape(-1), pk(x2, y2, z2), pln(x2, y2, z2),
            i3.reshape(-1), pk(x3, y3, z3), pln(x3, y3, z3))


def _knn_body(pt_ref, s_ref, nbr_ref, d2_ref, *, k, ns, tb, need_d2):
    pt = pt_ref[...]
    s = s_ref[...]
    xt = pt[:, 0:1]
    yt = pt[:, 1:2]
    zt = pt[:, 2:3]
    sx = s[0:1, :]
    sy = s[1:2, :]
    sz = s[2:3, :]
    nx = (sx * sx + sy * sy) + sz * sz
    ny = (xt * xt + yt * yt) + zt * zt
    mm = jnp.dot(pt, s, preferred_element_type=jnp.float32)
    d2q = (ny + nx) - 2.0 * mm
    d2q = jnp.maximum(d2q, 0.0)
    if need_d2:
        dx = xt - sx
        dy = yt - sy
        dz = zt - sz
        d2d = (dx * dx + dy * dy) + dz * dz
    lin = jax.lax.broadcasted_iota(jnp.int32, (tb, ns), 1)
    lane = jax.lax.broadcasted_iota(jnp.int32, (tb, 128), 1)
    nbr = jnp.zeros((tb, 128), jnp.int32)
    dsel = jnp.zeros((tb, 128), jnp.float32)
    for j in range(k):
        m = jnp.min(d2q, axis=1, keepdims=True)
        idx = jnp.min(jnp.where(d2q == m, lin, jnp.int32(2 ** 30)),
                      axis=1, keepdims=True)
        nbr = jnp.where(lane == j, idx, nbr)
        if need_d2:
            dj = jnp.min(jnp.where(lin == idx, d2d, jnp.inf),
                         axis=1, keepdims=True)
        else:
            dj = m
        dsel = jnp.where(lane == j, dj, dsel)
        d2q = jnp.where(lin == idx, jnp.inf, d2q)
    nbr_ref[...] = nbr
    d2_ref[...] = dsel


def _knn_pallas(pos_t, s_mat, k, need_d2=False):
    nt = pos_t.shape[0]
    ns = s_mat.shape[1]
    tb = min(nt, 128)
    body = functools.partial(_knn_body, k=k, ns=ns, tb=tb, need_d2=need_d2)
    nbr, d2 = pl.pallas_call(
        body,
        grid=(nt // tb,),
        in_specs=[
            pl.BlockSpec((tb, 3), lambda i: (i, 0)),
            pl.BlockSpec((3, ns), lambda i: (0, 0)),
        ],
        out_specs=[
            pl.BlockSpec((tb, 128), lambda i: (i, 0)),
            pl.BlockSpec((tb, 128), lambda i: (i, 0)),
        ],
        out_shape=[
            jax.ShapeDtypeStruct((nt, 128), jnp.int32),
            jax.ShapeDtypeStruct((nt, 128), jnp.float32),
        ],
    )(pos_t, s_mat)
    return nbr, d2


_NW = 32


def _sc_gather(table, idx):
    V, D = table.shape
    B = idx.shape[0]
    bpw = B // _NW
    chunk = max(cc for cc in range(8, min(bpw, 128) + 1, 8) if bpw % cc == 0)
    nchunks = bpw // chunk
    mesh = plsc.VectorSubcoreMesh(core_axis_name="c", subcore_axis_name="s")

    @functools.partial(
        pl.kernel, mesh=mesh,
        out_type=jax.ShapeDtypeStruct((B, D), jnp.float32),
        scratch_types=[
            pltpu.VMEM((bpw,), jnp.int32),
            pltpu.VMEM((chunk, D), jnp.float32),
            pltpu.VMEM((chunk, D), jnp.float32),
            pltpu.SemaphoreType.DMA,
            pltpu.SemaphoreType.DMA,
        ],
    )
    def k(table_hbm, idx_hbm, out_hbm, idx_v, rows_a, rows_b, gsem, osem):
        wid = jax.lax.axis_index("s") * 2 + jax.lax.axis_index("c")
        base = wid * bpw
        pltpu.sync_copy(idx_hbm.at[pl.ds(base, bpw)], idx_v)
        bufs = (rows_a, rows_b)
        outs = []
        for ci in range(nchunks):
            buf = bufs[ci % 2]
            if ci >= 2:
                outs[ci - 2].wait()
            pltpu.async_copy(
                table_hbm.at[idx_v.at[pl.ds(ci * chunk, chunk)]],
                buf, gsem).wait()
            outs.append(pltpu.async_copy(
                buf, out_hbm.at[pl.ds(base + ci * chunk, chunk)], osem))
        for cp in outs[max(0, nchunks - 2):]:
            cp.wait()

    return k(table, idx)


def _sa_body(g_ref, pt_ref, w1_ref, b1_ref, w2_ref, b2_ref, ws_ref, bs_ref,
             out_ref, *, kk, dx, dp, tb, dout):
    lane = jax.lax.broadcasted_iota(jnp.int32, (tb, dp), 1)
    px = pt_ref[:, 0:1]
    py = pt_ref[:, 1:2]
    pz = pt_ref[:, 2:3]
    posfull = (jnp.where(lane == dx, px, 0.0)
               + jnp.where(lane == dx + 1, py, 0.0)
               + jnp.where(lane == dx + 2, pz, 0.0))
    f32 = jnp.float32
    acc = jnp.full((tb, dout), -jnp.inf, f32)
    for j in range(kk):
        hj = g_ref[j] - posfull
        h1 = jnp.maximum(
            jnp.dot(hj, w1_ref[...], preferred_element_type=f32) + b1_ref[...],
            0.0)
        h2 = jnp.dot(h1, w2_ref[...], preferred_element_type=f32) + b2_ref[...]
        sc = jnp.dot(hj, ws_ref[...], preferred_element_type=f32) + bs_ref[...]
        acc = jnp.maximum(acc, jnp.maximum(h2 + sc, 0.0))
    out_ref[...] = acc


def _sa_pallas(g3, pos_t, p, dx):
    kk, nt, dp = g3.shape
    hid = p['w1'].shape[0]
    dout = p['w2'].shape[0]
    tb = min(nt, 128)
    w1t = jnp.zeros((dp, hid), jnp.float32).at[:dx + 3].set(p['w1'].T)
    wst = jnp.zeros((dp, dout), jnp.float32).at[:dx + 3].set(p['ws'].T)
    body = functools.partial(_sa_body, kk=kk, dx=dx, dp=dp, tb=tb, dout=dout)
    return pl.pallas_call(
        body,
        grid=(nt // tb,),
        in_specs=[
            pl.BlockSpec((kk, tb, dp), lambda i: (0, i, 0)),
            pl.BlockSpec((tb, 3), lambda i: (i, 0)),
            pl.BlockSpec((dp, hid), lambda i: (0, 0)),
            pl.BlockSpec((1, hid), lambda i: (0, 0)),
            pl.BlockSpec((hid, dout), lambda i: (0, 0)),
            pl.BlockSpec((1, dout), lambda i: (0, 0)),
            pl.BlockSpec((dp, dout), lambda i: (0, 0)),
            pl.BlockSpec((1, dout), lambda i: (0, 0)),
        ],
        out_specs=pl.BlockSpec((tb, dout), lambda i: (i, 0)),
        out_shape=jax.ShapeDtypeStruct((nt, dout), jnp.float32),
    )(g3, pos_t, w1t, p['b1'].reshape(1, -1), p['w2'].T,
      p['b2'].reshape(1, -1), wst, p['bs'].reshape(1, -1))


def _fp_body(g_ref, d2_ref, xt_ref, w1t_t_ref, w1t_s_ref, b1_ref,
             w2_ref, b2_ref, wst_t_ref, wst_s_ref, bs_ref, out_ref,
             *, has_xt, has_ws):
    w0 = 1.0 / jnp.maximum(d2_ref[:, 0:1], 1e-16)
    w1 = 1.0 / jnp.maximum(d2_ref[:, 1:2], 1e-16)
    w2 = 1.0 / jnp.maximum(d2_ref[:, 2:3], 1e-16)
    wsum = (w0 + w1) + w2
    interp = ((g_ref[0] * w0 + g_ref[1] * w1) + g_ref[2] * w2) / wsum
    f32 = jnp.float32
    h1 = jnp.dot(interp, w1t_s_ref[...], preferred_element_type=f32) + b1_ref[...]
    if has_xt:
        h1 = h1 + jnp.dot(xt_ref[...], w1t_t_ref[...], preferred_element_type=f32)
    h1 = jnp.maximum(h1, 0.0)
    h2 = jnp.dot(h1, w2_ref[...], preferred_element_type=f32) + b2_ref[...]
    if has_ws:
        sc = jnp.dot(interp, wst_s_ref[...], preferred_element_type=f32) + bs_ref[...]
        if has_xt:
            sc = sc + jnp.dot(xt_ref[...], wst_t_ref[...], preferred_element_type=f32)
    else:
        sc = interp
    out_ref[...] = jnp.maximum(h2 + sc, 0.0)


def _fp_pallas(g3, d2p, xt, p):
    _, nt, ds = g3.shape
    dt = 0 if xt is None else xt.shape[1]
    hid = p['w1'].shape[0]
    dout = p['w2'].shape[0]
    has_ws = 'ws' in p
    tb = min(nt, 256)
    w1t = p['w1'].T
    w1t_t, w1t_s = w1t[:dt], w1t[dt:]
    if has_ws:
        wst = p['ws'].T
        wst_t, wst_s = wst[:dt], wst[dt:]
        bs_in = p['bs'].reshape(1, -1)
    else:
        wst_t = jnp.zeros((dt, dout), jnp.float32)
        wst_s = jnp.zeros((ds, dout), jnp.float32)
        bs_in = jnp.zeros((1, dout), jnp.float32)
    if xt is None:
        dt_in = 8
        xt_in = jnp.zeros((nt, dt_in), jnp.float32)
        w1t_t = jnp.zeros((dt_in, hid), jnp.float32)
        wst_t = jnp.zeros((dt_in, dout), jnp.float32)
    else:
        xt_in = xt
        dt_in = dt
    body = functools.partial(_fp_body, has_xt=xt is not None, has_ws=has_ws)
    return pl.pallas_call(
        body,
        grid=(nt // tb,),
        in_specs=[
            pl.BlockSpec((3, tb, ds), lambda i: (0, i, 0)),
            pl.BlockSpec((tb, 128), lambda i: (i, 0)),
            pl.BlockSpec((tb, dt_in), lambda i: (i, 0)),
            pl.BlockSpec((dt_in, hid), lambda i: (0, 0)),
            pl.BlockSpec((ds, hid), lambda i: (0, 0)),
            pl.BlockSpec((1, hid), lambda i: (0, 0)),
            pl.BlockSpec((hid, dout), lambda i: (0, 0)),
            pl.BlockSpec((1, dout), lambda i: (0, 0)),
            pl.BlockSpec((dt_in, dout), lambda i: (0, 0)),
            pl.BlockSpec((ds, dout), lambda i: (0, 0)),
            pl.BlockSpec((1, dout), lambda i: (0, 0)),
        ],
        out_specs=pl.BlockSpec((tb, dout), lambda i: (i, 0)),
        out_shape=jax.ShapeDtypeStruct((nt, dout), jnp.float32),
    )(g3, d2p, xt_in, w1t_t, w1t_s, p['b1'].reshape(1, -1), p['w2'].T,
      p['b2'].reshape(1, -1), wst_t, wst_s, bs_in)


def _resmlp_body(x_ref, w1_ref, b1_ref, w2_ref, b2_ref, ws_ref, bs_ref,
                 out_ref):
    f32 = jnp.float32
    x = x_ref[...]
    h1 = jnp.maximum(
        jnp.dot(x, w1_ref[...], preferred_element_type=f32) + b1_ref[...], 0.0)
    h2 = jnp.dot(h1, w2_ref[...], preferred_element_type=f32) + b2_ref[...]
    sc = jnp.dot(x, ws_ref[...], preferred_element_type=f32) + bs_ref[...]
    out_ref[...] = jnp.maximum(h2 + sc, 0.0)


def _resmlp_pallas(x, p):
    n, _ = x.shape
    dout = p['w2'].shape[0]
    return pl.pallas_call(
        _resmlp_body,
        out_shape=jax.ShapeDtypeStruct((n, dout), jnp.float32),
    )(x, p['w1'].T, p['b1'].reshape(1, -1), p['w2'].T,
      p['b2'].reshape(1, -1), p['ws'].T, p['bs'].reshape(1, -1))


def _final_body(fin_ref, w1_ref, b1_ref, w2_ref, b2_ref, out_ref):
    h = jnp.maximum(fin_ref[...] @ w1_ref[...].T + b1_ref[...], 0.0)
    out_ref[...] = h @ w2_ref[...].T + b2_ref[...]


def _final_pallas(fin, w1, b1, w2, b2):
    n, d = fin.shape
    out_d = w2.shape[0]
    blk = 2048
    return pl.pallas_call(
        _final_body,
        grid=(n // blk,),
        in_specs=[
            pl.BlockSpec((blk, d), lambda i: (i, 0)),
            pl.BlockSpec((d, d), lambda i: (0, 0)),
            pl.BlockSpec((d,), lambda i: (0,)),
            pl.BlockSpec((out_d, d), lambda i: (0, 0)),
            pl.BlockSpec((out_d,), lambda i: (0,)),
        ],
        out_specs=pl.BlockSpec((blk, out_d), lambda i: (i, 0)),
        out_shape=jax.ShapeDtypeStruct((n, out_d), jnp.float32),
    )(fin, w1, b1, w2, b2)


def kernel(z, c, query_pos, query_pos_batch, params):
    pos0 = query_pos
    (idx1, pos1, pln1, idx2, pos2, pln2,
     idx3, pos3, pln3) = _fps_pallas(pos0)
    s0 = pos0.T
    s1 = jnp.concatenate(pln1, axis=0)
    s2 = jnp.concatenate(pln2, axis=0)
    s3 = jnp.concatenate(pln3, axis=0)
    nbr_sa1 = _knn_pallas(pos1, s0, K_SA)[0][:, :K_SA]
    nbr_sa2 = _knn_pallas(pos2, s1, K_SA)[0][:, :K_SA]
    nbr_sa3 = _knn_pallas(pos3, s2, K_SA)[0][:, :K_SA]
    batch3 = query_pos_batch[idx1][idx2][idx3]

    nbr_fp1, d2_fp1 = _knn_pallas(pos3, s3, K_FP, need_d2=True)
    nbr_fp2, d2_fp2 = _knn_pallas(pos2, s3, K_FP, need_d2=True)
    nbr_fp3, d2_fp3 = _knn_pallas(pos1, s2, K_FP, need_d2=True)
    nbr_fp4, d2_fp4 = _knn_pallas(pos0, s1, K_FP, need_d2=True)

    def src_self(nbr, nt):
        return jnp.concatenate([nbr.T.reshape(-1),
                                jnp.arange(nt, dtype=jnp.int32)])

    t1 = jnp.zeros((N_POINTS, 128), jnp.float32).at[:, :3].set(pos0)
    g1 = _sc_gather(t1, src_self(nbr_sa1, N1)).reshape(K_SA + 1, N1, 128)
    x1 = _sa_pallas(g1, pos1, params['sa1'], 0)
    t2 = jnp.concatenate([x1, pos1, jnp.zeros((N1, 61), jnp.float32)], axis=1)
    g2 = _sc_gather(t2, src_self(nbr_sa2, N2)).reshape(K_SA + 1, N2, 128)
    x2 = _sa_pallas(g2, pos2, params['sa2'], 64)
    t3 = jnp.concatenate([x2, pos2, jnp.zeros((N2, 125), jnp.float32)], axis=1)
    g3 = _sc_gather(t3, src_self(nbr_sa3, N3)).reshape(K_SA + 1, N3, 256)
    x3 = _sa_pallas(g3, pos3, params['sa3'], 128)

    bott_in = jnp.concatenate([x3, z[batch3], c[batch3]], axis=1)
    bott = _resmlp_pallas(bott_in, params['bottleneck'])

    def fp_stage(x_t, x_s, nbr, d2p, p):
        nt = nbr.shape[0]
        g = _sc_gather(x_s, nbr[:, :K_FP].T.reshape(-1))
        return _fp_pallas(g.reshape(K_FP, nt, x_s.shape[1]), d2p, x_t, p)

    up3 = fp_stage(x3, bott, nbr_fp1, d2_fp1, params['fp1'])
    up2 = fp_stage(x2, up3, nbr_fp2, d2_fp2, params['fp2'])
    up1 = fp_stage(x1, up2, nbr_fp3, d2_fp3, params['fp3'])
    fin = fp_stage(None, up1, nbr_fp4, d2_fp4, params['fp4'])
    return _final_pallas(fin, params['final_w1'], params['final_b1'],
                         params['final_w2'], params['final_b2'])


# speedup vs baseline: 10.2375x; 1.0009x over previous
import functools

import jax
import jax.numpy as jnp
from jax.experimental import pallas as pl
from jax.experimental.pallas import tpu as pltpu
from jax.experimental.pallas import tpu_sc as plsc

N_POINTS = 16384
N1, N2, N3 = 4096, 1024, 256
K_SA = 16
K_FP = 3


def _fps_level(X, Y, Z, n, m):
    cols = n // 8
    ocols = m // 8
    lin = (jax.lax.broadcasted_iota(jnp.int32, (8, cols), 0) * cols
           + jax.lax.broadcasted_iota(jnp.int32, (8, cols), 1))
    olin = (jax.lax.broadcasted_iota(jnp.int32, (8, ocols), 0) * ocols
            + jax.lax.broadcasted_iota(jnp.int32, (8, ocols), 1))
    x0 = X[0, 0]
    y0 = Y[0, 0]
    z0 = Z[0, 0]
    dx = X - x0
    dy = Y - y0
    dz = Z - z0
    dists = (dx * dx + dy * dy) + dz * dz
    zf = jnp.zeros((8, ocols), jnp.float32)
    idxp = jnp.zeros((8, ocols), jnp.int32)
    sel0 = olin == 0
    X1 = jnp.where(sel0, x0, zf)
    Y1 = jnp.where(sel0, y0, zf)
    Z1 = jnp.where(sel0, z0, zf)

    def body(i, st):
        dists, idxp, X1, Y1, Z1 = st
        mval = jnp.max(dists)
        nxt = jnp.min(jnp.where(dists == mval, lin, jnp.int32(2 ** 30)))
        sel = lin == nxt
        sx = jnp.sum(jnp.where(sel, X, 0.0))
        sy = jnp.sum(jnp.where(sel, Y, 0.0))
        sz = jnp.sum(jnp.where(sel, Z, 0.0))
        ddx = X - sx
        ddy = Y - sy
        ddz = Z - sz
        d = (ddx * ddx + ddy * ddy) + ddz * ddz
        dists = jnp.minimum(dists, d)
        om = olin == i
        idxp = jnp.where(om, nxt, idxp)
        X1 = jnp.where(om, sx, X1)
        Y1 = jnp.where(om, sy, Y1)
        Z1 = jnp.where(om, sz, Z1)
        return dists, idxp, X1, Y1, Z1

    _, idxp, X1, Y1, Z1 = jax.lax.fori_loop(
        1, m, body, (dists, idxp, X1, Y1, Z1))
    return idxp, X1, Y1, Z1


def _fps_body(px_ref, py_ref, pz_ref,
              i1_ref, x1_ref, y1_ref, z1_ref,
              i2_ref, x2_ref, y2_ref, z2_ref,
              i3_ref, x3_ref, y3_ref, z3_ref):
    X0, Y0, Z0 = px_ref[...], py_ref[...], pz_ref[...]
    i1, X1, Y1, Z1 = _fps_level(X0, Y0, Z0, N_POINTS, N1)
    i2, X2, Y2, Z2 = _fps_level(X1, Y1, Z1, N1, N2)
    i3, X3, Y3, Z3 = _fps_level(X2, Y2, Z2, N2, N3)
    i1_ref[...], x1_ref[...], y1_ref[...], z1_ref[...] = i1, X1, Y1, Z1
    i2_ref[...], x2_ref[...], y2_ref[...], z2_ref[...] = i2, X2, Y2, Z2
    i3_ref[...], x3_ref[...], y3_ref[...], z3_ref[...] = i3, X3, Y3, Z3


def _fps_pallas(pos0):
    planes = [pos0[:, c].reshape(8, -1) for c in range(3)]
    f32, i32 = jnp.float32, jnp.int32
    outs = pl.pallas_call(
        _fps_body,
        out_shape=[
            jax.ShapeDtypeStruct((8, N1 // 8), i32),
            jax.ShapeDtypeStruct((8, N1 // 8), f32),
            jax.ShapeDtypeStruct((8, N1 // 8), f32),
            jax.ShapeDtypeStruct((8, N1 // 8), f32),
            jax.ShapeDtypeStruct((8, N2 // 8), i32),
            jax.ShapeDtypeStruct((8, N2 // 8), f32),
            jax.ShapeDtypeStruct((8, N2 // 8), f32),
            jax.ShapeDtypeStruct((8, N2 // 8), f32),
            jax.ShapeDtypeStruct((8, N3 // 8), i32),
            jax.ShapeDtypeStruct((8, N3 // 8), f32),
            jax.ShapeDtypeStruct((8, N3 // 8), f32),
            jax.ShapeDtypeStruct((8, N3 // 8), f32),
        ],
    )(*planes)
    i1, x1, y1, z1, i2, x2, y2, z2, i3, x3, y3, z3 = outs

    def pk(x, y, z):
        return jnp.stack([x.reshape(-1), y.reshape(-1), z.reshape(-1)], axis=1)

    def pln(x, y, z):
        return (x.reshape(1, -1), y.reshape(1, -1), z.reshape(1, -1))

    return (i1.reshape(-1), pk(x1, y1, z1), pln(x1, y1, z1),
            i2.reshape(-1), pk(x2, y2, z2), pln(x2, y2, z2),
            i3.reshape(-1), pk(x3, y3, z3), pln(x3, y3, z3))


def _knn_body(pt_ref, s_ref, nbr_ref, d2_ref, *, k, ns, tb, need_d2):
    pt = pt_ref[...]
    s = s_ref[...]
    xt = pt[:, 0:1]
    yt = pt[:, 1:2]
    zt = pt[:, 2:3]
    sx = s[0:1, :]
    sy = s[1:2, :]
    sz = s[2:3, :]
    nx = (sx * sx + sy * sy) + sz * sz
    ny = (xt * xt + yt * yt) + zt * zt
    mm = jnp.dot(pt, s, preferred_element_type=jnp.float32)
    d2q = (ny + nx) - 2.0 * mm
    d2q = jnp.maximum(d2q, 0.0)
    if need_d2:
        dx = xt - sx
        dy = yt - sy
        dz = zt - sz
        d2d = (dx * dx + dy * dy) + dz * dz
    lin = jax.lax.broadcasted_iota(jnp.int32, (tb, ns), 1)
    lane = jax.lax.broadcasted_iota(jnp.int32, (tb, 128), 1)
    nbr = jnp.zeros((tb, 128), jnp.int32)
    dsel = jnp.zeros((tb, 128), jnp.float32)
    for j in range(k):
        m = jnp.min(d2q, axis=1, keepdims=True)
        idx = jnp.min(jnp.where(d2q == m, lin, jnp.int32(2 ** 30)),
                      axis=1, keepdims=True)
        nbr = jnp.where(lane == j, idx, nbr)
        if need_d2:
            dj = jnp.min(jnp.where(lin == idx, d2d, jnp.inf),
                         axis=1, keepdims=True)
        else:
            dj = m
        dsel = jnp.where(lane == j, dj, dsel)
        d2q = jnp.where(lin == idx, jnp.inf, d2q)
    nbr_ref[...] = nbr
    d2_ref[...] = dsel


def _knn_pallas(pos_t, s_mat, k, need_d2=False):
    nt = pos_t.shape[0]
    ns = s_mat.shape[1]
    tb = min(nt, 128)
    body = functools.partial(_knn_body, k=k, ns=ns, tb=tb, need_d2=need_d2)
    nbr, d2 = pl.pallas_call(
        body,
        grid=(nt // tb,),
        in_specs=[
            pl.BlockSpec((tb, 3), lambda i: (i, 0)),
            pl.BlockSpec((3, ns), lambda i: (0, 0)),
        ],
        out_specs=[
            pl.BlockSpec((tb, 128), lambda i: (i, 0)),
            pl.BlockSpec((tb, 128), lambda i: (i, 0)),
        ],
        out_shape=[
            jax.ShapeDtypeStruct((nt, 128), jnp.int32),
            jax.ShapeDtypeStruct((nt, 128), jnp.float32),
        ],
    )(pos_t, s_mat)
    return nbr, d2


_NW = 32


def _sc_gather(table, idx):
    V, D = table.shape
    B = idx.shape[0]
    bpw = B // _NW
    chunk = max(cc for cc in range(8, min(bpw, 128) + 1, 8) if bpw % cc == 0)
    nchunks = bpw // chunk
    mesh = plsc.VectorSubcoreMesh(core_axis_name="c", subcore_axis_name="s")

    @functools.partial(
        pl.kernel, mesh=mesh,
        out_type=jax.ShapeDtypeStruct((B, D), jnp.float32),
        scratch_types=[
            pltpu.VMEM((bpw,), jnp.int32),
            pltpu.VMEM((chunk, D), jnp.float32),
            pltpu.VMEM((chunk, D), jnp.float32),
            pltpu.SemaphoreType.DMA,
            pltpu.SemaphoreType.DMA,
        ],
    )
    def k(table_hbm, idx_hbm, out_hbm, idx_v, rows_a, rows_b, gsem, osem):
        wid = jax.lax.axis_index("s") * 2 + jax.lax.axis_index("c")
        base = wid * bpw
        pltpu.sync_copy(idx_hbm.at[pl.ds(base, bpw)], idx_v)
        bufs = (rows_a, rows_b)
        outs = []
        for ci in range(nchunks):
            buf = bufs[ci % 2]
            if ci >= 2:
                outs[ci - 2].wait()
            pltpu.async_copy(
                table_hbm.at[idx_v.at[pl.ds(ci * chunk, chunk)]],
                buf, gsem).wait()
            outs.append(pltpu.async_copy(
                buf, out_hbm.at[pl.ds(base + ci * chunk, chunk)], osem))
        for cp in outs[max(0, nchunks - 2):]:
            cp.wait()

    return k(table, idx)


def _sa_body(g_ref, pt_ref, w1_ref, b1_ref, w2_ref, b2_ref, ws_ref, bs_ref,
             out_ref, *, kk, dx, dp, tb, dout):
    lane = jax.lax.broadcasted_iota(jnp.int32, (tb, dp), 1)
    px = pt_ref[:, 0:1]
    py = pt_ref[:, 1:2]
    pz = pt_ref[:, 2:3]
    posfull = (jnp.where(lane == dx, px, 0.0)
               + jnp.where(lane == dx + 1, py, 0.0)
               + jnp.where(lane == dx + 2, pz, 0.0))
    f32 = jnp.float32
    acc = jnp.full((tb, dout), -jnp.inf, f32)
    for j in range(kk):
        hj = g_ref[j] - posfull
        h1 = jnp.maximum(
            jnp.dot(hj, w1_ref[...], preferred_element_type=f32) + b1_ref[...],
            0.0)
        h2 = jnp.dot(h1, w2_ref[...], preferred_element_type=f32) + b2_ref[...]
        sc = jnp.dot(hj, ws_ref[...], preferred_element_type=f32) + bs_ref[...]
        acc = jnp.maximum(acc, jnp.maximum(h2 + sc, 0.0))
    out_ref[...] = acc


def _sa_pallas(g3, pos_t, p, dx):
    kk, nt, dp = g3.shape
    hid = p['w1'].shape[0]
    dout = p['w2'].shape[0]
    tb = min(nt, 128)
    w1t = jnp.zeros((dp, hid), jnp.float32).at[:dx + 3].set(p['w1'].T)
    wst = jnp.zeros((dp, dout), jnp.float32).at[:dx + 3].set(p['ws'].T)
    body = functools.partial(_sa_body, kk=kk, dx=dx, dp=dp, tb=tb, dout=dout)
    return pl.pallas_call(
        body,
        grid=(nt // tb,),
        in_specs=[
            pl.BlockSpec((kk, tb, dp), lambda i: (0, i, 0)),
            pl.BlockSpec((tb, 3), lambda i: (i, 0)),
            pl.BlockSpec((dp, hid), lambda i: (0, 0)),
            pl.BlockSpec((1, hid), lambda i: (0, 0)),
            pl.BlockSpec((hid, dout), lambda i: (0, 0)),
            pl.BlockSpec((1, dout), lambda i: (0, 0)),
            pl.BlockSpec((dp, dout), lambda i: (0, 0)),
            pl.BlockSpec((1, dout), lambda i: (0, 0)),
        ],
        out_specs=pl.BlockSpec((tb, dout), lambda i: (i, 0)),
        out_shape=jax.ShapeDtypeStruct((nt, dout), jnp.float32),
    )(g3, pos_t, w1t, p['b1'].reshape(1, -1), p['w2'].T,
      p['b2'].reshape(1, -1), wst, p['bs'].reshape(1, -1))


def _fp_body(g_ref, d2_ref, xt_ref, w1t_t_ref, w1t_s_ref, b1_ref,
             w2_ref, b2_ref, wst_t_ref, wst_s_ref, bs_ref, *rest,
             has_xt, has_ws):
    if len(rest) == 1:
        fw1_ref = fb1_ref = fw2_ref = fb2_ref = None
        (out_ref,) = rest
    else:
        fw1_ref, fb1_ref, fw2_ref, fb2_ref, out_ref = rest
    w0 = 1.0 / jnp.maximum(d2_ref[:, 0:1], 1e-16)
    w1 = 1.0 / jnp.maximum(d2_ref[:, 1:2], 1e-16)
    w2 = 1.0 / jnp.maximum(d2_ref[:, 2:3], 1e-16)
    wsum = (w0 + w1) + w2
    interp = ((g_ref[0] * w0 + g_ref[1] * w1) + g_ref[2] * w2) / wsum
    f32 = jnp.float32
    h1 = jnp.dot(interp, w1t_s_ref[...], preferred_element_type=f32) + b1_ref[...]
    if has_xt:
        h1 = h1 + jnp.dot(xt_ref[...], w1t_t_ref[...], preferred_element_type=f32)
    h1 = jnp.maximum(h1, 0.0)
    h2 = jnp.dot(h1, w2_ref[...], preferred_element_type=f32) + b2_ref[...]
    if has_ws:
        sc = jnp.dot(interp, wst_s_ref[...], preferred_element_type=f32) + bs_ref[...]
        if has_xt:
            sc = sc + jnp.dot(xt_ref[...], wst_t_ref[...], preferred_element_type=f32)
    else:
        sc = interp
    out = jnp.maximum(h2 + sc, 0.0)
    if fw1_ref is not None:
        fh = jnp.maximum(
            jnp.dot(out, fw1_ref[...], preferred_element_type=f32)
            + fb1_ref[...], 0.0)
        out = jnp.dot(fh, fw2_ref[...], preferred_element_type=f32) + fb2_ref[...]
    out_ref[...] = out


def _fp_pallas(g3, d2p, xt, p, final=None):
    _, nt, ds = g3.shape
    dt = 0 if xt is None else xt.shape[1]
    hid = p['w1'].shape[0]
    dout = p['w2'].shape[0]
    has_ws = 'ws' in p
    tb = min(nt, 256)
    w1t = p['w1'].T
    w1t_t, w1t_s = w1t[:dt], w1t[dt:]
    if has_ws:
        wst = p['ws'].T
        wst_t, wst_s = wst[:dt], wst[dt:]
        bs_in = p['bs'].reshape(1, -1)
    else:
        wst_t = jnp.zeros((dt, dout), jnp.float32)
        wst_s = jnp.zeros((ds, dout), jnp.float32)
        bs_in = jnp.zeros((1, dout), jnp.float32)
    if xt is None:
        dt_in = 8
        xt_in = jnp.zeros((nt, dt_in), jnp.float32)
        w1t_t = jnp.zeros((dt_in, hid), jnp.float32)
        wst_t = jnp.zeros((dt_in, dout), jnp.float32)
    else:
        xt_in = xt
        dt_in = dt
    body = functools.partial(_fp_body, has_xt=xt is not None, has_ws=has_ws)
    in_specs = [
        pl.BlockSpec((3, tb, ds), lambda i: (0, i, 0)),
        pl.BlockSpec((tb, 128), lambda i: (i, 0)),
        pl.BlockSpec((tb, dt_in), lambda i: (i, 0)),
        pl.BlockSpec((dt_in, hid), lambda i: (0, 0)),
        pl.BlockSpec((ds, hid), lambda i: (0, 0)),
        pl.BlockSpec((1, hid), lambda i: (0, 0)),
        pl.BlockSpec((hid, dout), lambda i: (0, 0)),
        pl.BlockSpec((1, dout), lambda i: (0, 0)),
        pl.BlockSpec((dt_in, dout), lambda i: (0, 0)),
        pl.BlockSpec((ds, dout), lambda i: (0, 0)),
        pl.BlockSpec((1, dout), lambda i: (0, 0)),
    ]
    args = [g3, d2p, xt_in, w1t_t, w1t_s, p['b1'].reshape(1, -1), p['w2'].T,
            p['b2'].reshape(1, -1), wst_t, wst_s, bs_in]
    res_d = dout
    if final is not None:
        fw1, fb1, fw2, fb2 = final
        fh = fw1.shape[0]
        res_d = fw2.shape[0]
        in_specs += [
            pl.BlockSpec((dout, fh), lambda i: (0, 0)),
            pl.BlockSpec((1, fh), lambda i: (0, 0)),
            pl.BlockSpec((fh, res_d), lambda i: (0, 0)),
            pl.BlockSpec((1, res_d), lambda i: (0, 0)),
        ]
        args += [fw1.T, fb1.reshape(1, -1), fw2.T, fb2.reshape(1, -1)]
    return pl.pallas_call(
        body,
        grid=(nt // tb,),
        in_specs=in_specs,
        out_specs=pl.BlockSpec((tb, res_d), lambda i: (i, 0)),
        out_shape=jax.ShapeDtypeStruct((nt, res_d), jnp.float32),
    )(*args)


def _resmlp_body(x_ref, w1_ref, b1_ref, w2_ref, b2_ref, ws_ref, bs_ref,
                 out_ref):
    f32 = jnp.float32
    x = x_ref[...]
    h1 = jnp.maximum(
        jnp.dot(x, w1_ref[...], preferred_element_type=f32) + b1_ref[...], 0.0)
    h2 = jnp.dot(h1, w2_ref[...], preferred_element_type=f32) + b2_ref[...]
    sc = jnp.dot(x, ws_ref[...], preferred_element_type=f32) + bs_ref[...]
    out_ref[...] = jnp.maximum(h2 + sc, 0.0)


def _resmlp_pallas(x, p):
    n, _ = x.shape
    dout = p['w2'].shape[0]
    return pl.pallas_call(
        _resmlp_body,
        out_shape=jax.ShapeDtypeStruct((n, dout), jnp.float32),
    )(x, p['w1'].T, p['b1'].reshape(1, -1), p['w2'].T,
      p['b2'].reshape(1, -1), p['ws'].T, p['bs'].reshape(1, -1))


def _final_body(fin_ref, w1_ref, b1_ref, w2_ref, b2_ref, out_ref):
    h = jnp.maximum(fin_ref[...] @ w1_ref[...].T + b1_ref[...], 0.0)
    out_ref[...] = h @ w2_ref[...].T + b2_ref[...]


def _final_pallas(fin, w1, b1, w2, b2):
    n, d = fin.shape
    out_d = w2.shape[0]
    blk = 2048
    return pl.pallas_call(
        _final_body,
        grid=(n // blk,),
        in_specs=[
            pl.BlockSpec((blk, d), lambda i: (i, 0)),
            pl.BlockSpec((d, d), lambda i: (0, 0)),
            pl.BlockSpec((d,), lambda i: (0,)),
            pl.BlockSpec((out_d, d), lambda i: (0, 0)),
            pl.BlockSpec((out_d,), lambda i: (0,)),
        ],
        out_specs=pl.BlockSpec((blk, out_d), lambda i: (i, 0)),
        out_shape=jax.ShapeDtypeStruct((n, out_d), jnp.float32),
    )(fin, w1, b1, w2, b2)


def kernel(z, c, query_pos, query_pos_batch, params):
    pos0 = query_pos
    (idx1, pos1, pln1, idx2, pos2, pln2,
     idx3, pos3, pln3) = _fps_pallas(pos0)
    s0 = pos0.T
    s1 = jnp.concatenate(pln1, axis=0)
    s2 = jnp.concatenate(pln2, axis=0)
    s3 = jnp.concatenate(pln3, axis=0)
    nbr_sa1 = _knn_pallas(pos1, s0, K_SA)[0][:, :K_SA]
    nbr_sa2 = _knn_pallas(pos2, s1, K_SA)[0][:, :K_SA]
    nbr_sa3 = _knn_pallas(pos3, s2, K_SA)[0][:, :K_SA]
    batch3 = query_pos_batch[idx1][idx2][idx3]

    nbr_fp1, d2_fp1 = _knn_pallas(pos3, s3, K_FP, need_d2=True)
    nbr_fp2, d2_fp2 = _knn_pallas(pos2, s3, K_FP, need_d2=True)
    nbr_fp3, d2_fp3 = _knn_pallas(pos1, s2, K_FP, need_d2=True)
    nbr_fp4, d2_fp4 = _knn_pallas(pos0, s1, K_FP, need_d2=True)

    def src_self(nbr, nt):
        return jnp.concatenate([nbr.T.reshape(-1),
                                jnp.arange(nt, dtype=jnp.int32)])

    t1 = jnp.zeros((N_POINTS, 128), jnp.float32).at[:, :3].set(pos0)
    g1 = _sc_gather(t1, src_self(nbr_sa1, N1)).reshape(K_SA + 1, N1, 128)
    x1 = _sa_pallas(g1, pos1, params['sa1'], 0)
    t2 = jnp.concatenate([x1, pos1, jnp.zeros((N1, 61), jnp.float32)], axis=1)
    g2 = _sc_gather(t2, src_self(nbr_sa2, N2)).reshape(K_SA + 1, N2, 128)
    x2 = _sa_pallas(g2, pos2, params['sa2'], 64)
    t3 = jnp.concatenate([x2, pos2, jnp.zeros((N2, 125), jnp.float32)], axis=1)
    g3 = _sc_gather(t3, src_self(nbr_sa3, N3)).reshape(K_SA + 1, N3, 256)
    x3 = _sa_pallas(g3, pos3, params['sa3'], 128)

    bott_in = jnp.concatenate([x3, z[batch3], c[batch3]], axis=1)
    bott = _resmlp_pallas(bott_in, params['bottleneck'])

    def fp_stage(x_t, x_s, nbr, d2p, p, final=None):
        nt = nbr.shape[0]
        g = _sc_gather(x_s, nbr[:, :K_FP].T.reshape(-1))
        return _fp_pallas(g.reshape(K_FP, nt, x_s.shape[1]), d2p, x_t, p,
                          final=final)

    up3 = fp_stage(x3, bott, nbr_fp1, d2_fp1, params['fp1'])
    up2 = fp_stage(x2, up3, nbr_fp2, d2_fp2, params['fp2'])
    up1 = fp_stage(x1, up2, nbr_fp3, d2_fp3, params['fp3'])
    return fp_stage(None, up1, nbr_fp4, d2_fp4, params['fp4'],
                    final=(params['final_w1'], params['final_b1'],
                           params['final_w2'], params['final_b2']))
